# baseline (speedup 1.0000x reference)
import functools

import jax
import jax.numpy as jnp
from jax import lax
from jax.experimental import pallas as pl
from jax.experimental.pallas import tpu as pltpu
from jax.experimental.pallas import tpu_sc as plsc

NC = 2
NS = 16
NW = NC * NS


@functools.partial(jax.jit, static_argnums=(2, 3, 4))
def _lookup(table_flat, idx_flat, b, sp, d):
    mesh = plsc.VectorSubcoreMesh(core_axis_name="c", subcore_axis_name="s")
    s_len, p_len = 50, sp // 50
    feat = p_len * d
    bpw = b // NW
    n_grp = (sp + 15) // 16

    @functools.partial(
        pl.kernel,
        mesh=mesh,
        out_type=jax.ShapeDtypeStruct((b, s_len, feat), jnp.float32),
        scratch_types=[
            pltpu.VMEM((table_flat.shape[0],), jnp.float32),
            pltpu.VMEM((n_grp * 16,), jnp.int32),
            pltpu.VMEM((n_grp * 16,), jnp.int32),
            pltpu.VMEM((s_len, feat), jnp.float32),
            pltpu.VMEM((s_len, feat), jnp.float32),
            pltpu.SemaphoreType.DMA,
            pltpu.SemaphoreType.DMA,
            pltpu.SemaphoreType.DMA,
            pltpu.SemaphoreType.DMA,
        ],
        compiler_params=pltpu.CompilerParams(use_tc_tiling_on_sc=True,
                                             needs_layout_passes=False),
    )
    def k(table_hbm, idx_hbm, out_hbm, table_v, idxv0, idxv1, buf0, buf1,
          isem0, isem1, wsem0, wsem1):
        wid = lax.axis_index("s") * NC + lax.axis_index("c")
        b0 = wid * bpw
        pltpu.sync_copy(table_hbm, table_v)
        lanes = lax.iota(jnp.int32, 16)

        def idx_load(i, idxv, isem):
            pltpu.async_copy(idx_hbm.at[pl.ds((b0 + i) * sp, sp)],
                             idxv.at[pl.ds(0, sp)], isem)

        idx_load(0, idxv0, isem0)
        idx_load(1, idxv1, isem1)

        def do_batch(i, idxv, buf, isem, wsem):
            pltpu.make_async_copy(idx_hbm.at[pl.ds(b0 * sp, sp)],
                                  idxv.at[pl.ds(0, sp)], isem).wait()

            @pl.when(i >= 2)
            def _():
                pltpu.make_async_copy(buf, out_hbm.at[b0], wsem).wait()

            def grp(g, carry):
                s16, p16 = carry
                msk = g * 16 + lanes < sp
                idx16 = idxv[pl.ds(g * 16, 16)]
                ib = idx16 * d
                f16 = p16 * d
                for c in range(d):
                    vals = plsc.load_gather(table_v, [ib + c], mask=msk)
                    plsc.store_scatter(buf, [s16, f16 + c], vals, mask=msk)
                pn = p16 + 16
                wrap = pn >= p_len
                return (jnp.where(wrap, s16 + 1, s16),
                        jnp.where(wrap, pn - p_len, pn))

            lax.fori_loop(0, n_grp, grp,
                          (jnp.zeros((16,), jnp.int32), lanes))
            pltpu.async_copy(buf, out_hbm.at[b0 + i], wsem)

            @pl.when(i + 2 < bpw)
            def _():
                idx_load(i + 2, idxv, isem)

        def body(q, carry):
            do_batch(2 * q, idxv0, buf0, isem0, wsem0)
            do_batch(2 * q + 1, idxv1, buf1, isem1, wsem1)
            return carry

        lax.fori_loop(0, bpw // 2, body, 0)
        pltpu.make_async_copy(buf0, out_hbm.at[b0], wsem0).wait()
        pltpu.make_async_copy(buf1, out_hbm.at[b0], wsem1).wait()

    return k(table_flat, idx_flat)


def kernel(indices, table):
    b, s, p = indices.shape
    d = table.shape[1]
    return _lookup(table.reshape(-1),
                   indices.reshape(-1).astype(jnp.int32),
                   b, s * p, d)


# speedup vs baseline: 3.5385x; 3.5385x over previous
import functools

import jax
import jax.numpy as jnp
from jax import lax
from jax.experimental import pallas as pl
from jax.experimental.pallas import tpu as pltpu
from jax.experimental.pallas import tpu_sc as plsc

NC = 2
NS = 16
NW = NC * NS
WPB = 4


@functools.partial(jax.jit, static_argnums=(2, 3, 4))
def _lookup(table_flat, idx_flat, b, sp, d):
    mesh = plsc.VectorSubcoreMesh(core_axis_name="c", subcore_axis_name="s")
    s_len, p_len = 50, sp // 50
    feat = p_len * d
    bpw = b // NW
    sp_pad = (sp + 15) // 16 * 16

    @functools.partial(
        pl.kernel,
        mesh=mesh,
        out_type=jax.ShapeDtypeStruct((b, s_len, feat), jnp.float32),
        scratch_types=[
            pltpu.VMEM((table_flat.shape[0],), jnp.float32),
            pltpu.VMEM((sp_pad,), jnp.int32),
            pltpu.VMEM((sp_pad,), jnp.int32),
            pltpu.VMEM((s_len, feat), jnp.float32),
            pltpu.VMEM((s_len, feat), jnp.float32),
            pltpu.SemaphoreType.DMA,
            pltpu.SemaphoreType.DMA,
            pltpu.SemaphoreType.DMA,
            pltpu.SemaphoreType.DMA,
        ],
        compiler_params=pltpu.CompilerParams(use_tc_tiling_on_sc=True,
                                             needs_layout_passes=False),
    )
    def k(table_hbm, idx_hbm, out_hbm, table_v, idxv0, idxv1,
          buf0, buf1, isem0, isem1, wsem0, wsem1):
        wid = lax.axis_index("s") * NC + lax.axis_index("c")
        b0 = wid * bpw
        pltpu.sync_copy(table_hbm, table_v)

        def idx_load(i, idxv, isem):
            pltpu.async_copy(idx_hbm.at[pl.ds((b0 + i) * sp, sp)],
                             idxv.at[pl.ds(0, sp)], isem)

        idx_load(0, idxv0, isem0)
        idx_load(1, idxv1, isem1)

        def assemble_words(idxv, buf, srow0, q0, n_words):
            nv = (n_words * p_len + 15) // 16
            vs = [idxv[pl.ds(q0 + 16 * m, 16)] for m in range(nv)]
            for sr in range(n_words):
                for p in range(p_len):
                    q = sr * p_len + p
                    off = vs[q // 16][q % 16] * d
                    for h in range(d // 16):
                        buf[srow0 + sr, pl.ds(p * d + h * 16, 16)] = (
                            table_v[pl.ds(off + h * 16, 16)])

        def do_batch(i, idxv, buf, isem, wsem):
            pltpu.make_async_copy(idx_hbm.at[pl.ds(b0 * sp, sp)],
                                  idxv.at[pl.ds(0, sp)], isem).wait()

            @pl.when(i >= 2)
            def _():
                pltpu.make_async_copy(buf, out_hbm.at[b0], wsem).wait()

            def blk(t, carry):
                assemble_words(idxv, buf, t * WPB, t * WPB * p_len, WPB)
                return carry

            n_blk = s_len // WPB
            lax.fori_loop(0, n_blk, blk, 0)
            if s_len % WPB:
                assemble_words(idxv, buf, n_blk * WPB, n_blk * WPB * p_len,
                               s_len % WPB)
            pltpu.async_copy(buf, out_hbm.at[b0 + i], wsem)

            @pl.when(i + 2 < bpw)
            def _():
                idx_load(i + 2, idxv, isem)

        def body(q, carry):
            do_batch(2 * q, idxv0, buf0, isem0, wsem0)
            do_batch(2 * q + 1, idxv1, buf1, isem1, wsem1)
            return carry

        lax.fori_loop(0, bpw // 2, body, 0)
        pltpu.make_async_copy(buf0, out_hbm.at[b0], wsem0).wait()
        pltpu.make_async_copy(buf1, out_hbm.at[b0], wsem1).wait()

    return k(table_flat, idx_flat)


def kernel(indices, table):
    b, s, p = indices.shape
    d = table.shape[1]
    return _lookup(table.reshape(-1),
                   indices.reshape(-1).astype(jnp.int32),
                   b, s * p, d)


# speedup vs baseline: 3.5406x; 1.0006x over previous
import functools

import jax
import jax.numpy as jnp
from jax import lax
from jax.experimental import pallas as pl
from jax.experimental.pallas import tpu as pltpu
from jax.experimental.pallas import tpu_sc as plsc

NC = 2
NS = 16
NW = NC * NS
WPB = 4


@functools.partial(jax.jit, static_argnums=(2, 3, 4))
def _lookup(table_flat, idx_flat, b, sp, d):
    mesh = plsc.VectorSubcoreMesh(core_axis_name="c", subcore_axis_name="s")
    s_len, p_len = 50, sp // 50
    feat = p_len * d
    bpw = b // NW
    sp_pad = (sp + 15) // 16 * 16

    @functools.partial(
        pl.kernel,
        mesh=mesh,
        out_type=jax.ShapeDtypeStruct((b, s_len, feat), jnp.float32),
        scratch_types=[
            pltpu.VMEM((table_flat.shape[0],), jnp.float32),
            pltpu.VMEM((sp_pad,), jnp.int32),
            pltpu.VMEM((sp_pad,), jnp.int32),
            pltpu.VMEM((s_len, feat), jnp.float32),
            pltpu.VMEM((s_len, feat), jnp.float32),
            pltpu.SemaphoreType.DMA,
            pltpu.SemaphoreType.DMA,
            pltpu.SemaphoreType.DMA,
            pltpu.SemaphoreType.DMA,
        ],
        compiler_params=pltpu.CompilerParams(use_tc_tiling_on_sc=True,
                                             needs_layout_passes=False),
    )
    def k(table_hbm, idx_hbm, out_hbm, table_v, idxv0, idxv1,
          buf0, buf1, isem0, isem1, wsem0, wsem1):
        wid = lax.axis_index("s") * NC + lax.axis_index("c")
        b0 = wid * bpw
        pltpu.sync_copy(table_hbm, table_v)

        def idx_load(i, idxv, isem):
            pltpu.async_copy(idx_hbm.at[pl.ds((b0 + i) * sp, sp)],
                             idxv.at[pl.ds(0, sp)], isem)

        idx_load(0, idxv0, isem0)
        idx_load(1, idxv1, isem1)

        def assemble_words(idxv, buf, srow0, q0, n_words):
            nv = (n_words * p_len + 15) // 16
            vs = [idxv[pl.ds(q0 + 16 * m, 16)] * d for m in range(nv)]
            for sr in range(n_words):
                for p in range(p_len):
                    q = sr * p_len + p
                    off = vs[q // 16][q % 16]
                    for h in range(d // 16):
                        buf[srow0 + sr, pl.ds(p * d + h * 16, 16)] = (
                            table_v[pl.ds(off + h * 16, 16)])

        def do_batch(i, idxv, buf, isem, wsem):
            pltpu.make_async_copy(idx_hbm.at[pl.ds(b0 * sp, sp)],
                                  idxv.at[pl.ds(0, sp)], isem).wait()

            @pl.when(i >= 2)
            def _():
                pltpu.make_async_copy(buf, out_hbm.at[b0], wsem).wait()

            def blk(t, carry):
                assemble_words(idxv, buf, t * WPB, t * WPB * p_len, WPB)
                return carry

            n_blk = s_len // WPB
            lax.fori_loop(0, n_blk, blk, 0)
            if s_len % WPB:
                assemble_words(idxv, buf, n_blk * WPB, n_blk * WPB * p_len,
                               s_len % WPB)
            pltpu.async_copy(buf, out_hbm.at[b0 + i], wsem)

            @pl.when(i + 2 < bpw)
            def _():
                idx_load(i + 2, idxv, isem)

        def body(q, carry):
            do_batch(2 * q, idxv0, buf0, isem0, wsem0)
            do_batch(2 * q + 1, idxv1, buf1, isem1, wsem1)
            return carry

        lax.fori_loop(0, bpw // 2, body, 0)
        pltpu.make_async_copy(buf0, out_hbm.at[b0], wsem0).wait()
        pltpu.make_async_copy(buf1, out_hbm.at[b0], wsem1).wait()

    return k(table_flat, idx_flat)


def kernel(indices, table):
    b, s, p = indices.shape
    d = table.shape[1]
    return _lookup(table.reshape(-1),
                   indices.reshape(-1).astype(jnp.int32),
                   b, s * p, d)


# speedup vs baseline: 3.8219x; 1.0794x over previous
import functools

import jax
import jax.numpy as jnp
from jax import lax
from jax.experimental import pallas as pl
from jax.experimental.pallas import tpu as pltpu
from jax.experimental.pallas import tpu_sc as plsc

NC = 2
NS = 16
NW = NC * NS
CHUNK = 128
K = 5


@functools.partial(jax.jit, static_argnums=(2, 3))
def _gather_rows(table, idx, n_chunks, d):
    mesh = plsc.VectorSubcoreMesh(core_axis_name="c", subcore_axis_name="s")
    n_super = n_chunks // K
    assert n_chunks % K == 0 and n_super % 2 == 0

    @functools.partial(
        pl.kernel,
        mesh=mesh,
        out_type=jax.ShapeDtypeStruct((NW * n_super, K, CHUNK, d),
                                      jnp.float32),
        scratch_types=[
            pltpu.VMEM((n_chunks, CHUNK), jnp.int32),
            pltpu.VMEM((K, CHUNK, d), jnp.float32),
            pltpu.VMEM((K, CHUNK, d), jnp.float32),
            pltpu.VMEM_SHARED((103, d), jnp.float32),
            pltpu.SemaphoreType.DMA,
            pltpu.SemaphoreType.DMA,
            pltpu.SemaphoreType.DMA,
        ],
        compiler_params=pltpu.CompilerParams(use_tc_tiling_on_sc=False),
    )
    def k(table_hbm, idx_hbm, out_hbm, idx_v, buf0, buf1, table_v, gsem,
          wsem0, wsem1):
        wid = lax.axis_index("s") * NC + lax.axis_index("c")
        base = wid * n_super
        @pl.when(lax.axis_index("s") == 0)
        def _():
            pltpu.sync_copy(table_hbm, table_v)
        pltpu.sync_copy(idx_hbm.at[pl.ds(wid * n_chunks, n_chunks)], idx_v)
        plsc.subcore_barrier()

        def super_chunk(sc, buf, wsem, first):
            @pl.when(jnp.logical_not(first))
            def _():
                pltpu.make_async_copy(buf, out_hbm.at[base], wsem).wait()
            cps = []
            for t in range(K):
                cps.append(pltpu.async_copy(
                    table_v.at[idx_v.at[sc * K + t]], buf.at[t], gsem))
            for cp in cps:
                cp.wait()
            pltpu.async_copy(buf, out_hbm.at[base + sc], wsem)

        def body(p, carry):
            super_chunk(2 * p, buf0, wsem0, p == 0)
            super_chunk(2 * p + 1, buf1, wsem1, p == 0)
            return carry

        lax.fori_loop(0, n_super // 2, body, 0)
        pltpu.make_async_copy(buf0, out_hbm.at[base], wsem0).wait()
        pltpu.make_async_copy(buf1, out_hbm.at[base], wsem1).wait()

    return k(table, idx)


def kernel(indices, table):
    b, s, p = indices.shape
    d = table.shape[1]
    total = b * s * p
    n_chunks = total // (NW * CHUNK)
    idx = indices.reshape(NW * n_chunks, CHUNK).astype(jnp.int32)
    out = _gather_rows(table, idx, n_chunks, d)
    return out.reshape(b, s, p * d)
